# baseline (speedup 1.0000x reference)
import functools

import jax
import jax.numpy as jnp
from jax import lax
from jax.experimental import pallas as pl
from jax.experimental.pallas import tpu as pltpu
from jax.experimental.pallas import tpu_sc as plsc

_N = 10000
_D = 128
_H = 128
_RB = 400

_NP = 10240
_NT = 16
_RT = _NP // _NT
_NA = 10112
_RA = _NA // _NT
_CH = 128
_ECT = 81
_NSEG = 3
_SCH = _ECT // _NSEG
_EP = 2 * _NT * _ECT * _CH


def _deg_body(dst_hbm, w_hbm, zer_hbm, out_hbm, dst_v, w_v, acc_sh):
    cid = lax.axis_index("c")
    sid = lax.axis_index("s")
    wid = cid * _NT + sid
    pltpu.sync_copy(dst_hbm.at[wid], dst_v)
    pltpu.sync_copy(w_hbm.at[wid], w_v)
    pltpu.sync_copy(zer_hbm, acc_sh.at[pl.ds(sid * _RT, _RT)])
    plsc.subcore_barrier()

    def body(j, carry):
        pltpu.sync_copy(w_v.at[j], acc_sh.at[dst_v.at[j]], add=True)
        return carry

    lax.fori_loop(0, _ECT, body, 0)
    plsc.subcore_barrier()
    pltpu.sync_copy(acc_sh.at[pl.ds(sid * _RT, _RT)],
                    out_hbm.at[pl.ds(cid * _NP + sid * _RT, _RT)])


_deg_kernel = functools.partial(
    pl.kernel,
    out_type=jax.ShapeDtypeStruct((2 * _NP,), jnp.float32),
    mesh=plsc.VectorSubcoreMesh(core_axis_name="c", subcore_axis_name="s"),
    scratch_types=[
        pltpu.VMEM((_ECT, _CH), jnp.int32),
        pltpu.VMEM((_ECT, _CH), jnp.float32),
        pltpu.VMEM_SHARED((_NP,), jnp.float32),
    ],
)(_deg_body)


def _msg_body(src_hbm, dst_hbm, w_hbm, tab_hbm, zer_hbm,
              s0_hbm, s1_hbm, s2_hbm,
              src_v, dst_v, w_v, bufa, bufb, acc_sh, gsa, gsb, ssa, ssb):
    cid = lax.axis_index("c")
    sid = lax.axis_index("s")
    wid = cid * _NT + sid
    pltpu.sync_copy(zer_hbm, acc_sh.at[pl.ds(sid * _RA, _RA)])

    def stage_segment(seg, off):
        pltpu.sync_copy(src_hbm.at[wid, seg], src_v)
        pltpu.sync_copy(dst_hbm.at[wid, seg], dst_v)
        pltpu.sync_copy(w_hbm.at[wid, seg], w_v)
        if off:
            def pre_body(j, carry):
                for k in range(_CH // 16):
                    src_v[j, pl.ds(k * 16, 16)] = (
                        src_v[j, pl.ds(k * 16, 16)] + off)
                return carry

            lax.fori_loop(0, _SCH, pre_body, 0)

    def run_segment(seg, off):
        stage_segment(seg, off)
        g_start(0, bufa)
        lax.fori_loop(0, _SCH // 2, pair_body, 0)
        g_wait(bufa)
        mul(bufa, _SCH - 1)
        s_start(_SCH - 1, bufa)
        s_wait(bufa)
        s_wait(bufb)

    def g_start(j, buf):
        sem = gsa if buf is bufa else gsb
        pltpu.async_copy(tab_hbm.at[src_v.at[j]], buf, sem)

    def g_wait(buf):
        sem = gsa if buf is bufa else gsb
        pltpu.make_async_copy(tab_hbm.at[pl.ds(0, _CH)], buf, sem).wait()

    def s_start(j, buf):
        sem = ssa if buf is bufa else ssb
        pltpu.async_copy(buf, acc_sh.at[dst_v.at[j]], sem, add=True)

    def s_wait(buf):
        sem = ssa if buf is bufa else ssb
        pltpu.make_async_copy(buf, acc_sh.at[pl.ds(0, _CH)], sem).wait()

    def mul(buf, j):
        def mul_body(kk, carry2):
            wv = w_v[j, pl.ds(kk * 16, 16)]
            for l in range(16):
                wk = wv[l]
                e = kk * 16 + l
                for c in range(_H // 16):
                    buf[e, pl.ds(c * 16, 16)] = buf[e, pl.ds(c * 16, 16)] * wk
            return carry2

        lax.fori_loop(0, _CH // 16, mul_body, 0)

    def writeback(dst_hbm_ref):
        pltpu.sync_copy(acc_sh.at[pl.ds(sid * _RA, _RA)],
                        dst_hbm_ref.at[pl.ds(cid * _NA + sid * _RA, _RA)])

    def pair_body(jj, carry):
        a = 2 * jj
        b = a + 1

        @pl.when(jj > 0)
        def _():
            s_wait(bufb)

        g_start(b, bufb)
        g_wait(bufa)
        mul(bufa, a)
        s_start(a, bufa)
        g_wait(bufb)
        mul(bufb, b)
        s_start(b, bufb)
        s_wait(bufa)

        @pl.when(b + 1 < _SCH)
        def _():
            g_start(b + 1, bufa)

        return carry

    for g, sg_hbm in enumerate((s0_hbm, s1_hbm, s2_hbm)):
        if g > 0:
            pltpu.sync_copy(zer_hbm, acc_sh.at[pl.ds(sid * _RA, _RA)])
        plsc.subcore_barrier()

        def seg_body(seg, carry, _off=g * _N):
            run_segment(seg, _off)
            return carry

        lax.fori_loop(0, _NSEG, seg_body, 0)
        plsc.subcore_barrier()
        writeback(sg_hbm)
        plsc.subcore_barrier()


_msg_kernel = functools.partial(
    pl.kernel,
    out_type=[jax.ShapeDtypeStruct((2 * _NA, _H), jnp.float32),
              jax.ShapeDtypeStruct((2 * _NA, _H), jnp.float32),
              jax.ShapeDtypeStruct((2 * _NA, _H), jnp.float32)],
    mesh=plsc.VectorSubcoreMesh(core_axis_name="c", subcore_axis_name="s"),
    scratch_types=[
        pltpu.VMEM((_SCH, _CH), jnp.int32),
        pltpu.VMEM((_SCH, _CH), jnp.int32),
        pltpu.VMEM((_SCH, _CH), jnp.float32),
        pltpu.VMEM((_CH, _H), jnp.float32),
        pltpu.VMEM((_CH, _H), jnp.float32),
        pltpu.VMEM_SHARED((_NA, _H), jnp.float32),
        pltpu.SemaphoreType.DMA,
        pltpu.SemaphoreType.DMA,
        pltpu.SemaphoreType.DMA,
        pltpu.SemaphoreType.DMA,
    ],
)(_msg_body)


def _mm_body(x_ref, w_ref, d0_ref, d1_ref, o_ref, dinv_ref):
    xw = jnp.dot(x_ref[...], w_ref[...], preferred_element_type=jnp.float32)
    dinv = lax.rsqrt(d0_ref[...] + d1_ref[...])
    y = xw * dinv
    o_ref[0] = y[:, :_H]
    o_ref[1] = y[:, _H:2 * _H]
    o_ref[2] = y[:, 2 * _H:]
    dinv_ref[...] = dinv


def _fused_xw(x, wcat, deg0, deg1):
    grid = _N // _RB
    return pl.pallas_call(
        _mm_body,
        grid=(grid,),
        in_specs=[
            pl.BlockSpec((_RB, _D), lambda i: (i, 0)),
            pl.BlockSpec((_D, 3 * _H), lambda i: (0, 0)),
            pl.BlockSpec((_RB, 1), lambda i: (i, 0)),
            pl.BlockSpec((_RB, 1), lambda i: (i, 0)),
        ],
        out_specs=[
            pl.BlockSpec((3, _RB, _H), lambda i: (0, i, 0)),
            pl.BlockSpec((_RB, 1), lambda i: (i, 0)),
        ],
        out_shape=[
            jax.ShapeDtypeStruct((3, _N, _H), jnp.float32),
            jax.ShapeDtypeStruct((_N, 1), jnp.float32),
        ],
    )(x, wcat, deg0.reshape(_NP, 1)[:_N], deg1.reshape(_NP, 1)[:_N])


def _gates_body(a0_ref, a0b_ref, a1_ref, a1b_ref, a2_ref, a2b_ref, dinv_ref, h_ref,
                wlz_ref, wlr_ref, wlh_ref,
                wlin_ref, bcat_ref, blz_ref, blr_ref, blh_ref, blin_ref,
                out_ref, h0_ref):
    conv = jnp.concatenate(
        [a0_ref[0] + a0b_ref[0], a1_ref[0] + a1b_ref[0],
         a2_ref[0] + a2b_ref[0]], axis=1)
    conv = conv * dinv_ref[...] + bcat_ref[...]
    cz = conv[:, :_H]
    cr = conv[:, _H:2 * _H]
    ch = conv[:, 2 * _H:]
    hh = h_ref[...]
    wlz = wlz_ref[...]
    wlr = wlr_ref[...]
    wlh = wlh_ref[...]

    def mm(a, b):
        return jnp.dot(a, b, preferred_element_type=jnp.float32)

    z = jax.nn.sigmoid(mm(cz, wlz[:_H]) + mm(hh, wlz[_H:]) + blz_ref[...])
    r = jax.nn.sigmoid(mm(cr, wlr[:_H]) + mm(hh, wlr[_H:]) + blr_ref[...])
    ht = jnp.tanh(mm(ch, wlh[:_H]) + mm(hh * r, wlh[_H:]) + blh_ref[...])
    h0 = z * hh + (1.0 - z) * ht
    out_ref[...] = mm(jax.nn.relu(h0), wlin_ref[...]) + blin_ref[...]
    h0_ref[...] = h0


def _gates(s0, s1, s2p, dinv, h,
           wlz, wlr, wlh, wlin, bcat, blz, blr, blh, blin):
    grid = _N // _RB
    full = lambda shape: pl.BlockSpec(shape, lambda i: tuple(0 for _ in shape))
    return pl.pallas_call(
        _gates_body,
        grid=(grid,),
        in_specs=[
            pl.BlockSpec((1, _RB, _H), lambda i: (0, i, 0)),
            pl.BlockSpec((1, _RB, _H), lambda i: (1, i, 0)),
            pl.BlockSpec((1, _RB, _H), lambda i: (0, i, 0)),
            pl.BlockSpec((1, _RB, _H), lambda i: (1, i, 0)),
            pl.BlockSpec((1, _RB, _H), lambda i: (0, i, 0)),
            pl.BlockSpec((1, _RB, _H), lambda i: (1, i, 0)),
            pl.BlockSpec((_RB, 1), lambda i: (i, 0)),
            pl.BlockSpec((_RB, _H), lambda i: (i, 0)),
            full((2 * _H, _H)),
            full((2 * _H, _H)),
            full((2 * _H, _H)),
            full((_H, _H)),
            full((1, 3 * _H)),
            full((1, _H)),
            full((1, _H)),
            full((1, _H)),
            full((1, _H)),
        ],
        out_specs=[
            pl.BlockSpec((_RB, _H), lambda i: (i, 0)),
            pl.BlockSpec((_RB, _H), lambda i: (i, 0)),
        ],
        out_shape=[
            jax.ShapeDtypeStruct((_N, _H), jnp.float32),
            jax.ShapeDtypeStruct((_N, _H), jnp.float32),
        ],
    )(s0, s0, s1, s1, s2p, s2p, dinv, h, wlz, wlr, wlh, wlin, bcat,
      blz[None, :], blr[None, :], blh[None, :], blin[None, :])


def kernel(x, edge_index, edge_weight, h, Wz, bz, Wr, br, Wh, bh,
           Wlz, blz, Wlr, blr, Wlh, blh, Wlin, blin):
    wcat = jnp.concatenate([Wz, Wr, Wh], axis=1)
    bcat = jnp.concatenate([bz, br, bh])[None, :]

    src = edge_index[0]
    dst = edge_index[1]
    loop = jnp.arange(_N, dtype=src.dtype)
    npad = _EP - src.shape[0] - _N
    izer = jnp.zeros((npad,), src.dtype)
    s2 = jnp.concatenate([src, loop, izer])
    d2 = jnp.concatenate([dst, loop, izer])
    w2 = jnp.concatenate([edge_weight, jnp.ones((_N,), jnp.float32),
                          jnp.zeros((npad,), jnp.float32)])
    src3 = s2.reshape(2 * _NT, _ECT, _CH)
    dst3 = d2.reshape(2 * _NT, _ECT, _CH)
    w3 = w2.reshape(2 * _NT, _ECT, _CH)
    zer = jnp.zeros((_RT,), jnp.float32)
    zer2 = jnp.zeros((_RA, _H), jnp.float32)

    degout = _deg_kernel(dst3, w3, zer)
    a, dinv = _fused_xw(x, wcat, degout[:_NP], degout[_NP:])
    tab = a.reshape(3 * _N, _H)
    src4 = s2.reshape(2 * _NT, _NSEG, _SCH, _CH)
    dst4 = d2.reshape(2 * _NT, _NSEG, _SCH, _CH)
    w4 = w2.reshape(2 * _NT, _NSEG, _SCH, _CH)
    s0p, s1p, s2p = _msg_kernel(src4, dst4, w4, tab, zer2)

    out, h0 = _gates(s0p.reshape(2, _NA, _H), s1p.reshape(2, _NA, _H),
                     s2p.reshape(2, _NA, _H),
                     dinv, h, Wlz, Wlr, Wlh, Wlin,
                     bcat, blz, blr, blh, blin)
    return (out, h0)


# speedup vs baseline: 1.0792x; 1.0792x over previous
import functools

import jax
import jax.numpy as jnp
from jax import lax
from jax.experimental import pallas as pl
from jax.experimental.pallas import tpu as pltpu
from jax.experimental.pallas import tpu_sc as plsc

_N = 10000
_D = 128
_H = 128
_RB = 400

_NP = 10240
_NT = 16
_RT = _NP // _NT
_NA = 10112
_RA = _NA // _NT
_CH = 128
_ECT = 81
_MCH = 96
_NSEG = 4
_SCH = 27
_EP = 2 * _NT * _ECT * _CH


def _deg_body(dst_hbm, w_hbm, zer_hbm, out_hbm, dst_v, w_v, acc_sh):
    cid = lax.axis_index("c")
    sid = lax.axis_index("s")
    wid = cid * _NT + sid
    pltpu.sync_copy(dst_hbm.at[wid], dst_v)
    pltpu.sync_copy(w_hbm.at[wid], w_v)
    pltpu.sync_copy(zer_hbm, acc_sh.at[pl.ds(sid * _RT, _RT)])
    plsc.subcore_barrier()

    def body(j, carry):
        pltpu.sync_copy(w_v.at[j], acc_sh.at[dst_v.at[j]], add=True)
        return carry

    lax.fori_loop(0, _ECT, body, 0)
    plsc.subcore_barrier()
    pltpu.sync_copy(acc_sh.at[pl.ds(sid * _RT, _RT)],
                    out_hbm.at[pl.ds(cid * _NP + sid * _RT, _RT)])


_deg_kernel = functools.partial(
    pl.kernel,
    out_type=jax.ShapeDtypeStruct((2 * _NP,), jnp.float32),
    mesh=plsc.VectorSubcoreMesh(core_axis_name="c", subcore_axis_name="s"),
    scratch_types=[
        pltpu.VMEM((_ECT, _CH), jnp.int32),
        pltpu.VMEM((_ECT, _CH), jnp.float32),
        pltpu.VMEM_SHARED((_NP,), jnp.float32),
    ],
)(_deg_body)


def _msg_body(src_hbm, dst_hbm, w_hbm, tab_hbm, zer_hbm,
              s0_hbm, s1_hbm, s2_hbm,
              src_v, dst_v, w_v, bufa, bufb, bufc, acc_sh,
              gsa, gsb, gsc, ssa, ssb, ssc):
    cid = lax.axis_index("c")
    sid = lax.axis_index("s")
    wid = cid * _NT + sid
    pltpu.sync_copy(zer_hbm, acc_sh.at[pl.ds(sid * _RA, _RA)])

    def stage_segment(seg, off):
        pltpu.sync_copy(src_hbm.at[wid, seg], src_v)
        pltpu.sync_copy(dst_hbm.at[wid, seg], dst_v)
        pltpu.sync_copy(w_hbm.at[wid, seg], w_v)
        if off:
            def pre_body(j, carry):
                for k in range(_MCH // 16):
                    src_v[j, pl.ds(k * 16, 16)] = (
                        src_v[j, pl.ds(k * 16, 16)] + off)
                return carry

            lax.fori_loop(0, _SCH, pre_body, 0)

    bufs = (bufa, bufb, bufc)
    gsems = (gsa, gsb, gsc)
    ssems = (ssa, ssb, ssc)

    def g_start(j, k):
        pltpu.async_copy(tab_hbm.at[src_v.at[j]], bufs[k], gsems[k])

    def g_wait(k):
        pltpu.make_async_copy(tab_hbm.at[pl.ds(0, _MCH)], bufs[k],
                              gsems[k]).wait()

    def s_start(j, k):
        pltpu.async_copy(bufs[k], acc_sh.at[dst_v.at[j]], ssems[k], add=True)

    def s_wait(k):
        pltpu.make_async_copy(bufs[k], acc_sh.at[pl.ds(0, _MCH)],
                              ssems[k]).wait()

    def mul(k, j):
        buf = bufs[k]

        def mul_body(kk, carry2):
            wv = w_v[j, pl.ds(kk * 16, 16)]
            for l in range(16):
                wk = wv[l]
                e = kk * 16 + l
                for c in range(_H // 16):
                    buf[e, pl.ds(c * 16, 16)] = buf[e, pl.ds(c * 16, 16)] * wk
            return carry2

        lax.fori_loop(0, _MCH // 16, mul_body, 0)

    def run_segment(seg, off):
        stage_segment(seg, off)
        g_start(0, 0)
        g_start(1, 1)

        def trip_body(tt, carry):
            for k in range(3):
                c = 3 * tt + k
                g_wait(k)
                mul(k, c)
                s_start(c, k)
                kn = (k + 2) % 3
                if k == 0:
                    @pl.when(tt > 0)
                    def _():
                        s_wait(kn)
                        g_start(c + 2, kn)

                    @pl.when(tt == 0)
                    def _():
                        g_start(c + 2, kn)
                elif k == 1:
                    s_wait(kn)

                    @pl.when(tt < (_SCH // 3) - 1)
                    def _():
                        g_start(c + 2, kn)
                else:
                    s_wait(kn)

                    @pl.when(tt < (_SCH // 3) - 1)
                    def _():
                        g_start(c + 2, kn)
            return carry

        lax.fori_loop(0, _SCH // 3, trip_body, 0)
        s_wait(2)

    def writeback(dst_hbm_ref):
        pltpu.sync_copy(acc_sh.at[pl.ds(sid * _RA, _RA)],
                        dst_hbm_ref.at[pl.ds(cid * _NA + sid * _RA, _RA)])

    for g, sg_hbm in enumerate((s0_hbm, s1_hbm, s2_hbm)):
        if g > 0:
            pltpu.sync_copy(zer_hbm, acc_sh.at[pl.ds(sid * _RA, _RA)])
        plsc.subcore_barrier()

        def seg_body(seg, carry, _off=g * _N):
            run_segment(seg, _off)
            return carry

        lax.fori_loop(0, _NSEG, seg_body, 0)
        plsc.subcore_barrier()
        writeback(sg_hbm)
        plsc.subcore_barrier()


_msg_kernel = functools.partial(
    pl.kernel,
    out_type=[jax.ShapeDtypeStruct((2 * _NA, _H), jnp.float32),
              jax.ShapeDtypeStruct((2 * _NA, _H), jnp.float32),
              jax.ShapeDtypeStruct((2 * _NA, _H), jnp.float32)],
    mesh=plsc.VectorSubcoreMesh(core_axis_name="c", subcore_axis_name="s"),
    scratch_types=[
        pltpu.VMEM((_SCH, _MCH), jnp.int32),
        pltpu.VMEM((_SCH, _MCH), jnp.int32),
        pltpu.VMEM((_SCH, _MCH), jnp.float32),
        pltpu.VMEM((_MCH, _H), jnp.float32),
        pltpu.VMEM((_MCH, _H), jnp.float32),
        pltpu.VMEM((_MCH, _H), jnp.float32),
        pltpu.VMEM_SHARED((_NA, _H), jnp.float32),
        pltpu.SemaphoreType.DMA,
        pltpu.SemaphoreType.DMA,
        pltpu.SemaphoreType.DMA,
        pltpu.SemaphoreType.DMA,
        pltpu.SemaphoreType.DMA,
        pltpu.SemaphoreType.DMA,
    ],
)(_msg_body)


def _mm_body(x_ref, w_ref, d0_ref, d1_ref, o_ref, dinv_ref):
    xw = jnp.dot(x_ref[...], w_ref[...], preferred_element_type=jnp.float32)
    dinv = lax.rsqrt(d0_ref[...] + d1_ref[...])
    y = xw * dinv
    o_ref[0] = y[:, :_H]
    o_ref[1] = y[:, _H:2 * _H]
    o_ref[2] = y[:, 2 * _H:]
    dinv_ref[...] = dinv


def _fused_xw(x, wcat, deg0, deg1):
    grid = _N // _RB
    return pl.pallas_call(
        _mm_body,
        grid=(grid,),
        in_specs=[
            pl.BlockSpec((_RB, _D), lambda i: (i, 0)),
            pl.BlockSpec((_D, 3 * _H), lambda i: (0, 0)),
            pl.BlockSpec((_RB, 1), lambda i: (i, 0)),
            pl.BlockSpec((_RB, 1), lambda i: (i, 0)),
        ],
        out_specs=[
            pl.BlockSpec((3, _RB, _H), lambda i: (0, i, 0)),
            pl.BlockSpec((_RB, 1), lambda i: (i, 0)),
        ],
        out_shape=[
            jax.ShapeDtypeStruct((3, _N, _H), jnp.float32),
            jax.ShapeDtypeStruct((_N, 1), jnp.float32),
        ],
    )(x, wcat, deg0.reshape(_NP, 1)[:_N], deg1.reshape(_NP, 1)[:_N])


def _gates_body(a0_ref, a0b_ref, a1_ref, a1b_ref, a2_ref, a2b_ref, dinv_ref, h_ref,
                wlz_ref, wlr_ref, wlh_ref,
                wlin_ref, bcat_ref, blz_ref, blr_ref, blh_ref, blin_ref,
                out_ref, h0_ref):
    conv = jnp.concatenate(
        [a0_ref[0] + a0b_ref[0], a1_ref[0] + a1b_ref[0],
         a2_ref[0] + a2b_ref[0]], axis=1)
    conv = conv * dinv_ref[...] + bcat_ref[...]
    cz = conv[:, :_H]
    cr = conv[:, _H:2 * _H]
    ch = conv[:, 2 * _H:]
    hh = h_ref[...]
    wlz = wlz_ref[...]
    wlr = wlr_ref[...]
    wlh = wlh_ref[...]

    def mm(a, b):
        return jnp.dot(a, b, preferred_element_type=jnp.float32)

    z = jax.nn.sigmoid(mm(cz, wlz[:_H]) + mm(hh, wlz[_H:]) + blz_ref[...])
    r = jax.nn.sigmoid(mm(cr, wlr[:_H]) + mm(hh, wlr[_H:]) + blr_ref[...])
    ht = jnp.tanh(mm(ch, wlh[:_H]) + mm(hh * r, wlh[_H:]) + blh_ref[...])
    h0 = z * hh + (1.0 - z) * ht
    out_ref[...] = mm(jax.nn.relu(h0), wlin_ref[...]) + blin_ref[...]
    h0_ref[...] = h0


def _gates(s0, s1, s2p, dinv, h,
           wlz, wlr, wlh, wlin, bcat, blz, blr, blh, blin):
    grid = _N // _RB
    full = lambda shape: pl.BlockSpec(shape, lambda i: tuple(0 for _ in shape))
    return pl.pallas_call(
        _gates_body,
        grid=(grid,),
        in_specs=[
            pl.BlockSpec((1, _RB, _H), lambda i: (0, i, 0)),
            pl.BlockSpec((1, _RB, _H), lambda i: (1, i, 0)),
            pl.BlockSpec((1, _RB, _H), lambda i: (0, i, 0)),
            pl.BlockSpec((1, _RB, _H), lambda i: (1, i, 0)),
            pl.BlockSpec((1, _RB, _H), lambda i: (0, i, 0)),
            pl.BlockSpec((1, _RB, _H), lambda i: (1, i, 0)),
            pl.BlockSpec((_RB, 1), lambda i: (i, 0)),
            pl.BlockSpec((_RB, _H), lambda i: (i, 0)),
            full((2 * _H, _H)),
            full((2 * _H, _H)),
            full((2 * _H, _H)),
            full((_H, _H)),
            full((1, 3 * _H)),
            full((1, _H)),
            full((1, _H)),
            full((1, _H)),
            full((1, _H)),
        ],
        out_specs=[
            pl.BlockSpec((_RB, _H), lambda i: (i, 0)),
            pl.BlockSpec((_RB, _H), lambda i: (i, 0)),
        ],
        out_shape=[
            jax.ShapeDtypeStruct((_N, _H), jnp.float32),
            jax.ShapeDtypeStruct((_N, _H), jnp.float32),
        ],
    )(s0, s0, s1, s1, s2p, s2p, dinv, h, wlz, wlr, wlh, wlin, bcat,
      blz[None, :], blr[None, :], blh[None, :], blin[None, :])


def kernel(x, edge_index, edge_weight, h, Wz, bz, Wr, br, Wh, bh,
           Wlz, blz, Wlr, blr, Wlh, blh, Wlin, blin):
    wcat = jnp.concatenate([Wz, Wr, Wh], axis=1)
    bcat = jnp.concatenate([bz, br, bh])[None, :]

    src = edge_index[0]
    dst = edge_index[1]
    loop = jnp.arange(_N, dtype=src.dtype)
    npad = _EP - src.shape[0] - _N
    izer = jnp.zeros((npad,), src.dtype)
    s2 = jnp.concatenate([src, loop, izer])
    d2 = jnp.concatenate([dst, loop, izer])
    w2 = jnp.concatenate([edge_weight, jnp.ones((_N,), jnp.float32),
                          jnp.zeros((npad,), jnp.float32)])
    src3 = s2.reshape(2 * _NT, _ECT, _CH)
    dst3 = d2.reshape(2 * _NT, _ECT, _CH)
    w3 = w2.reshape(2 * _NT, _ECT, _CH)
    zer = jnp.zeros((_RT,), jnp.float32)
    zer2 = jnp.zeros((_RA, _H), jnp.float32)

    degout = _deg_kernel(dst3, w3, zer)
    a, dinv = _fused_xw(x, wcat, degout[:_NP], degout[_NP:])
    tab = a.reshape(3 * _N, _H)
    src4 = s2.reshape(2 * _NT, _NSEG, _SCH, _MCH)
    dst4 = d2.reshape(2 * _NT, _NSEG, _SCH, _MCH)
    w4 = w2.reshape(2 * _NT, _NSEG, _SCH, _MCH)
    s0p, s1p, s2p = _msg_kernel(src4, dst4, w4, tab, zer2)

    out, h0 = _gates(s0p.reshape(2, _NA, _H), s1p.reshape(2, _NA, _H),
                     s2p.reshape(2, _NA, _H),
                     dinv, h, Wlz, Wlr, Wlh, Wlin,
                     bcat, blz, blr, blh, blin)
    return (out, h0)
